# baseline (speedup 1.0000x reference)
import functools

import jax
import jax.numpy as jnp
from jax import lax
from jax.experimental import pallas as pl
from jax.experimental.pallas import tpu as pltpu
from jax.experimental.pallas import tpu_sc as plsc

B = 1024
S = 2048
E = 16
P = 512
TB = 256
NSLOT = 6144
NBLK = 23
NBLK_PAD = 24

NC, NS = 2, 16
NW = NC * NS
TPW = B // NW


def _router_body(x_ref, wg_ref, bg_ref, slot1_ref, slot2_ref, w1_ref, w2_ref,
                 bexp_ref, nb_ref):
    g = jnp.dot(x_ref[...], wg_ref[...], preferred_element_type=jnp.float32)
    g = g + bg_ref[...]
    iota_e = lax.broadcasted_iota(jnp.int32, (B, E), 1)
    m1 = jnp.max(g, axis=1, keepdims=True)
    a1 = jnp.min(jnp.where(g == m1, iota_e, E), axis=1, keepdims=True)
    gm = jnp.where(iota_e == a1, -jnp.inf, g)
    m2 = jnp.max(gm, axis=1, keepdims=True)
    a2 = jnp.min(jnp.where(gm == m2, iota_e, E), axis=1, keepdims=True)
    w1 = 1.0 / (1.0 + jnp.exp(m2 - m1))
    w1_ref[...] = jnp.broadcast_to(w1, (B, E))
    w2_ref[...] = jnp.broadcast_to(1.0 - w1, (B, E))

    oh1 = (iota_e == a1).astype(jnp.float32)
    oh2 = (iota_e == a2).astype(jnp.float32)
    ib_r = lax.broadcasted_iota(jnp.int32, (B, B), 0)
    ib_c = lax.broadcasted_iota(jnp.int32, (B, B), 1)
    ltri = (ib_r > ib_c).astype(jnp.float32)
    ex1 = jnp.dot(ltri, oh1, preferred_element_type=jnp.float32)
    ex2 = jnp.dot(ltri, oh2, preferred_element_type=jnp.float32)
    counts1 = jnp.sum(oh1, axis=0, keepdims=True)
    counts2 = jnp.sum(oh2, axis=0, keepdims=True)
    counts = counts1 + counts2
    rank1 = jnp.sum(ex1 * oh1, axis=1, keepdims=True)
    rank2 = (jnp.sum(ex2 * oh2, axis=1, keepdims=True)
             + jnp.sum(counts1 * oh2, axis=1, keepdims=True))
    padded = jnp.ceil(counts * (1.0 / TB)) * TB
    ie_r = lax.broadcasted_iota(jnp.int32, (E, E), 0)
    ie_c = lax.broadcasted_iota(jnp.int32, (E, E), 1)
    offs = jnp.sum(jnp.where(ie_r < ie_c, jnp.broadcast_to(padded.reshape(E, 1), (E, E)), 0.0),
                   axis=0, keepdims=True)
    ends = offs + padded
    slot1_ref[...] = (jnp.sum(offs * oh1, axis=1, keepdims=True)
                      + rank1).astype(jnp.int32).reshape(B)
    slot2_ref[...] = (jnp.sum(offs * oh2, axis=1, keepdims=True)
                      + rank2).astype(jnp.int32).reshape(B)
    jblk = (lax.broadcasted_iota(jnp.int32, (NBLK_PAD, E), 0) * TB).astype(jnp.float32)
    bexp = jnp.sum((jnp.broadcast_to(ends, (NBLK_PAD, E)) <= jblk).astype(jnp.int32),
                   axis=1, keepdims=True)
    bexp_ref[...] = jnp.minimum(bexp, E - 1).reshape(NBLK_PAD)
    nb_ref[...] = (jnp.sum(padded, axis=1, keepdims=True) * (1.0 / TB)).astype(jnp.int32)


def _router_call(x, Wg, bg2):
    return pl.pallas_call(
        _router_body,
        out_shape=[
            jax.ShapeDtypeStruct((B,), jnp.int32),
            jax.ShapeDtypeStruct((B,), jnp.int32),
            jax.ShapeDtypeStruct((B, E), jnp.float32),
            jax.ShapeDtypeStruct((B, E), jnp.float32),
            jax.ShapeDtypeStruct((NBLK_PAD,), jnp.int32),
            jax.ShapeDtypeStruct((1, 1), jnp.int32),
        ],
    )(x, Wg, bg2)


HALF = TPW // 2


def _dispatch_body(x_hbm, slot1_hbm, slot2_hbm, xs_hbm, idx1a_v, idx1b_v,
                   idx2a_v, idx2b_v, rows_v, sem1, sem2, sem3, sem4):
    wid = lax.axis_index("s") * NC + lax.axis_index("c")
    base = wid * TPW
    pltpu.sync_copy(slot1_hbm.at[pl.ds(base, HALF)], idx1a_v)
    pltpu.sync_copy(slot2_hbm.at[pl.ds(base, HALF)], idx2a_v)
    pltpu.sync_copy(slot1_hbm.at[pl.ds(base + HALF, HALF)], idx1b_v)
    pltpu.sync_copy(slot2_hbm.at[pl.ds(base + HALF, HALF)], idx2b_v)
    pltpu.sync_copy(x_hbm.at[pl.ds(base, HALF)], rows_v.at[pl.ds(0, HALF)])
    d1 = pltpu.async_copy(rows_v.at[pl.ds(0, HALF)], xs_hbm.at[idx1a_v], sem1)
    d2 = pltpu.async_copy(rows_v.at[pl.ds(0, HALF)], xs_hbm.at[idx2a_v], sem2)
    pltpu.sync_copy(x_hbm.at[pl.ds(base + HALF, HALF)],
                    rows_v.at[pl.ds(HALF, HALF)])
    d3 = pltpu.async_copy(rows_v.at[pl.ds(HALF, HALF)], xs_hbm.at[idx1b_v], sem3)
    d4 = pltpu.async_copy(rows_v.at[pl.ds(HALF, HALF)], xs_hbm.at[idx2b_v], sem4)
    d1.wait()
    d2.wait()
    d3.wait()
    d4.wait()


@functools.cache
def _dispatch_call():
    mesh = plsc.VectorSubcoreMesh(core_axis_name="c", subcore_axis_name="s",
                                  num_cores=NC, num_subcores=NS)
    return pl.kernel(
        _dispatch_body,
        out_type=jax.ShapeDtypeStruct((NSLOT, S), jnp.float32),
        mesh=mesh,
        scratch_types=[
            pltpu.VMEM((HALF,), jnp.int32),
            pltpu.VMEM((HALF,), jnp.int32),
            pltpu.VMEM((HALF,), jnp.int32),
            pltpu.VMEM((HALF,), jnp.int32),
            pltpu.VMEM((TPW, S), jnp.float32),
            pltpu.SemaphoreType.DMA,
            pltpu.SemaphoreType.DMA,
            pltpu.SemaphoreType.DMA,
            pltpu.SemaphoreType.DMA,
        ],
    )


def _mm_body(nb_ref, bexp_ref, xs_ref, we_ref, be_ref, y_ref):
    j = pl.program_id(0)

    @pl.when(j < nb_ref[0])
    def _():
        y_ref[...] = (jnp.dot(xs_ref[...], we_ref[0],
                              preferred_element_type=jnp.float32)
                      + be_ref[0])


def _mm_call(nb, bexp, xs, We, be):
    def _j(j, nb_ref):
        return jnp.minimum(j, nb_ref[0] - 1)

    grid_spec = pltpu.PrefetchScalarGridSpec(
        num_scalar_prefetch=2,
        grid=(NBLK,),
        in_specs=[
            pl.BlockSpec((TB, S), lambda j, nb, bexp: (_j(j, nb), 0)),
            pl.BlockSpec((1, S, P), lambda j, nb, bexp: (bexp[_j(j, nb)], 0, 0)),
            pl.BlockSpec((1, 1, P), lambda j, nb, bexp: (bexp[_j(j, nb)], 0, 0)),
        ],
        out_specs=pl.BlockSpec((TB, P), lambda j, nb, bexp: (_j(j, nb), 0)),
    )
    return pl.pallas_call(
        _mm_body,
        grid_spec=grid_spec,
        out_shape=jax.ShapeDtypeStruct((NSLOT, P), jnp.float32),
    )(nb, bexp, xs, We, be.reshape(E, 1, P))


def _combine_body(y_hbm, slot1_hbm, slot2_hbm, w1_hbm, w2_hbm, out_hbm,
                  idx1_v, idx2_v, w1_v, w2_v, r1_v, r2_v, o_v, sem1, sem2):
    wid = lax.axis_index("s") * NC + lax.axis_index("c")
    base = wid * TPW
    pltpu.sync_copy(slot1_hbm.at[pl.ds(base, TPW)], idx1_v)
    pltpu.sync_copy(slot2_hbm.at[pl.ds(base, TPW)], idx2_v)
    pltpu.sync_copy(w1_hbm.at[pl.ds(base, TPW)], w1_v)
    pltpu.sync_copy(w2_hbm.at[pl.ds(base, TPW)], w2_v)
    g1 = pltpu.async_copy(y_hbm.at[idx1_v], r1_v, sem1)
    g2 = pltpu.async_copy(y_hbm.at[idx2_v], r2_v, sem2)
    g1.wait()
    g2.wait()

    def body(i, carry):
        w1b = w1_v[i, :]
        w2b = w2_v[i, :]
        for c in range(P // 16):
            sl = pl.ds(c * 16, 16)
            o_v[i, sl] = w1b * r1_v[i, sl] + w2b * r2_v[i, sl]
        return carry

    lax.fori_loop(0, TPW, body, 0)
    pltpu.sync_copy(o_v, out_hbm.at[pl.ds(base, TPW)])


@functools.cache
def _combine_call():
    mesh = plsc.VectorSubcoreMesh(core_axis_name="c", subcore_axis_name="s",
                                  num_cores=NC, num_subcores=NS)
    return pl.kernel(
        _combine_body,
        out_type=jax.ShapeDtypeStruct((B, P), jnp.float32),
        mesh=mesh,
        scratch_types=[
            pltpu.VMEM((TPW,), jnp.int32),
            pltpu.VMEM((TPW,), jnp.int32),
            pltpu.VMEM((TPW, E), jnp.float32),
            pltpu.VMEM((TPW, E), jnp.float32),
            pltpu.VMEM((TPW, P), jnp.float32),
            pltpu.VMEM((TPW, P), jnp.float32),
            pltpu.VMEM((TPW, P), jnp.float32),
            pltpu.SemaphoreType.DMA,
            pltpu.SemaphoreType.DMA,
        ],
    )


def kernel(x, Wg, bg, We, be):
    slot1, slot2, w1c, w2c, bexp, nbc = _router_call(x, Wg, bg.reshape(1, E))
    nb = nbc.reshape(1)
    xs = _dispatch_call()(x, slot1, slot2)
    ys = _mm_call(nb, bexp, xs, We, be)
    return _combine_call()(ys, slot1, slot2, w1c, w2c)


# speedup vs baseline: 1.0189x; 1.0189x over previous
import functools

import jax
import jax.numpy as jnp
from jax import lax
from jax.experimental import pallas as pl
from jax.experimental.pallas import tpu as pltpu
from jax.experimental.pallas import tpu_sc as plsc

B = 1024
S = 2048
E = 16
P = 512
TB = 256
NSLOT = 6144
NBLK = 23
NBLK_PAD = 24

NC, NS = 2, 16
NW = NC * NS
TPW = B // NW


def _router_body(x_ref, wg_ref, bg_ref, slot1_ref, slot2_ref, w1_ref, w2_ref,
                 bexp_ref, nb_ref):
    g = jnp.dot(x_ref[...], wg_ref[...], preferred_element_type=jnp.float32)
    g = g + bg_ref[...]
    iota_e = lax.broadcasted_iota(jnp.int32, (B, E), 1)
    m1 = jnp.max(g, axis=1, keepdims=True)
    a1 = jnp.min(jnp.where(g == m1, iota_e, E), axis=1, keepdims=True)
    gm = jnp.where(iota_e == a1, -jnp.inf, g)
    m2 = jnp.max(gm, axis=1, keepdims=True)
    a2 = jnp.min(jnp.where(gm == m2, iota_e, E), axis=1, keepdims=True)
    w1 = 1.0 / (1.0 + jnp.exp(m2 - m1))
    w1_ref[...] = jnp.broadcast_to(w1, (B, E))
    w2_ref[...] = jnp.broadcast_to(1.0 - w1, (B, E))

    oh1 = (iota_e == a1).astype(jnp.float32)
    oh2 = (iota_e == a2).astype(jnp.float32)
    ib_r = lax.broadcasted_iota(jnp.int32, (B, B), 0)
    ib_c = lax.broadcasted_iota(jnp.int32, (B, B), 1)
    ltri = (ib_r > ib_c).astype(jnp.float32)
    ex1 = jnp.dot(ltri, oh1, preferred_element_type=jnp.float32)
    ex2 = jnp.dot(ltri, oh2, preferred_element_type=jnp.float32)
    counts1 = jnp.sum(oh1, axis=0, keepdims=True)
    counts2 = jnp.sum(oh2, axis=0, keepdims=True)
    counts = counts1 + counts2
    rank1 = jnp.sum(ex1 * oh1, axis=1, keepdims=True)
    rank2 = (jnp.sum(ex2 * oh2, axis=1, keepdims=True)
             + jnp.sum(counts1 * oh2, axis=1, keepdims=True))
    padded = jnp.ceil(counts * (1.0 / TB)) * TB
    ie_r = lax.broadcasted_iota(jnp.int32, (E, E), 0)
    ie_c = lax.broadcasted_iota(jnp.int32, (E, E), 1)
    offs = jnp.sum(jnp.where(ie_r < ie_c, jnp.broadcast_to(padded.reshape(E, 1), (E, E)), 0.0),
                   axis=0, keepdims=True)
    ends = offs + padded
    slot1_ref[...] = (jnp.sum(offs * oh1, axis=1, keepdims=True)
                      + rank1).astype(jnp.int32).reshape(B)
    slot2_ref[...] = (jnp.sum(offs * oh2, axis=1, keepdims=True)
                      + rank2).astype(jnp.int32).reshape(B)
    jblk = (lax.broadcasted_iota(jnp.int32, (NBLK_PAD, E), 0) * TB).astype(jnp.float32)
    bexp = jnp.sum((jnp.broadcast_to(ends, (NBLK_PAD, E)) <= jblk).astype(jnp.int32),
                   axis=1, keepdims=True)
    bexp_ref[...] = jnp.minimum(bexp, E - 1).reshape(NBLK_PAD)
    nb_ref[...] = (jnp.sum(padded, axis=1, keepdims=True) * (1.0 / TB)).astype(jnp.int32)


def _router_call(x, Wg, bg2):
    return pl.pallas_call(
        _router_body,
        out_shape=[
            jax.ShapeDtypeStruct((B,), jnp.int32),
            jax.ShapeDtypeStruct((B,), jnp.int32),
            jax.ShapeDtypeStruct((B, E), jnp.float32),
            jax.ShapeDtypeStruct((B, E), jnp.float32),
            jax.ShapeDtypeStruct((NBLK_PAD,), jnp.int32),
            jax.ShapeDtypeStruct((1, 1), jnp.int32),
        ],
    )(x, Wg, bg2)


def _dispatch_body(x_hbm, slot1_hbm, slot2_hbm, xs_hbm, idx1_v, idx2_v,
                   rows_v, sem1, sem2):
    wid = lax.axis_index("s") * NC + lax.axis_index("c")
    base = wid * TPW
    pltpu.sync_copy(slot1_hbm.at[pl.ds(base, TPW)], idx1_v)
    pltpu.sync_copy(slot2_hbm.at[pl.ds(base, TPW)], idx2_v)
    pltpu.sync_copy(x_hbm.at[pl.ds(base, TPW)], rows_v)
    d1 = pltpu.async_copy(rows_v, xs_hbm.at[idx1_v], sem1)
    d2 = pltpu.async_copy(rows_v, xs_hbm.at[idx2_v], sem2)
    d1.wait()
    d2.wait()


@functools.cache
def _dispatch_call():
    mesh = plsc.VectorSubcoreMesh(core_axis_name="c", subcore_axis_name="s",
                                  num_cores=NC, num_subcores=NS)
    return pl.kernel(
        _dispatch_body,
        out_type=jax.ShapeDtypeStruct((NSLOT, S), jnp.float32),
        mesh=mesh,
        scratch_types=[
            pltpu.VMEM((TPW,), jnp.int32),
            pltpu.VMEM((TPW,), jnp.int32),
            pltpu.VMEM((TPW, S), jnp.float32),
            pltpu.SemaphoreType.DMA,
            pltpu.SemaphoreType.DMA,
        ],
    )


def _mm_body(nb_ref, bexp_ref, xs_ref, we_ref, be_ref, y_ref):
    j = pl.program_id(0)

    @pl.when(j < nb_ref[0])
    def _():
        y_ref[...] = (jnp.dot(xs_ref[...], we_ref[0],
                              preferred_element_type=jnp.float32)
                      + be_ref[0])


def _mm_call(nb, bexp, xs, We, be):
    def _j(j, nb_ref):
        return jnp.minimum(j, nb_ref[0] - 1)

    grid_spec = pltpu.PrefetchScalarGridSpec(
        num_scalar_prefetch=2,
        grid=(NBLK,),
        in_specs=[
            pl.BlockSpec((TB, S), lambda j, nb, bexp: (_j(j, nb), 0)),
            pl.BlockSpec((1, S, P), lambda j, nb, bexp: (bexp[_j(j, nb)], 0, 0)),
            pl.BlockSpec((1, 1, P), lambda j, nb, bexp: (bexp[_j(j, nb)], 0, 0)),
        ],
        out_specs=pl.BlockSpec((TB, P), lambda j, nb, bexp: (_j(j, nb), 0)),
    )
    return pl.pallas_call(
        _mm_body,
        grid_spec=grid_spec,
        out_shape=jax.ShapeDtypeStruct((NSLOT, P), jnp.float32),
    )(nb, bexp, xs, We, be.reshape(E, 1, P))


def _combine_body(y_hbm, slot1_hbm, slot2_hbm, w1_hbm, w2_hbm, out_hbm,
                  idx1_v, idx2_v, w1_v, w2_v, r1_v, r2_v, o_v, sem1, sem2):
    wid = lax.axis_index("s") * NC + lax.axis_index("c")
    base = wid * TPW
    pltpu.sync_copy(slot1_hbm.at[pl.ds(base, TPW)], idx1_v)
    pltpu.sync_copy(slot2_hbm.at[pl.ds(base, TPW)], idx2_v)
    pltpu.sync_copy(w1_hbm.at[pl.ds(base, TPW)], w1_v)
    pltpu.sync_copy(w2_hbm.at[pl.ds(base, TPW)], w2_v)
    g1 = pltpu.async_copy(y_hbm.at[idx1_v], r1_v, sem1)
    g2 = pltpu.async_copy(y_hbm.at[idx2_v], r2_v, sem2)
    g1.wait()
    g2.wait()

    def body(i, carry):
        w1b = w1_v[i, :]
        w2b = w2_v[i, :]
        for c in range(P // 16):
            sl = pl.ds(c * 16, 16)
            o_v[i, sl] = w1b * r1_v[i, sl] + w2b * r2_v[i, sl]
        return carry

    lax.fori_loop(0, TPW, body, 0)
    pltpu.sync_copy(o_v, out_hbm.at[pl.ds(base, TPW)])


@functools.cache
def _combine_call():
    mesh = plsc.VectorSubcoreMesh(core_axis_name="c", subcore_axis_name="s",
                                  num_cores=NC, num_subcores=NS)
    return pl.kernel(
        _combine_body,
        out_type=jax.ShapeDtypeStruct((B, P), jnp.float32),
        mesh=mesh,
        scratch_types=[
            pltpu.VMEM((TPW,), jnp.int32),
            pltpu.VMEM((TPW,), jnp.int32),
            pltpu.VMEM((TPW, E), jnp.float32),
            pltpu.VMEM((TPW, E), jnp.float32),
            pltpu.VMEM((TPW, P), jnp.float32),
            pltpu.VMEM((TPW, P), jnp.float32),
            pltpu.VMEM((TPW, P), jnp.float32),
            pltpu.SemaphoreType.DMA,
            pltpu.SemaphoreType.DMA,
        ],
    )


def kernel(x, Wg, bg, We, be):
    slot1, slot2, w1c, w2c, bexp, nbc = _router_call(x, Wg, bg.reshape(1, E))
    nb = nbc.reshape(1)
    xs = _dispatch_call()(x, slot1, slot2)
    ys = _mm_call(nb, bexp, xs, We, be)
    return _combine_call()(ys, slot1, slot2, w1c, w2c)


# speedup vs baseline: 1.0264x; 1.0074x over previous
import functools

import jax
import jax.numpy as jnp
from jax import lax
from jax.experimental import pallas as pl
from jax.experimental.pallas import tpu as pltpu
from jax.experimental.pallas import tpu_sc as plsc

B = 1024
S = 2048
E = 16
P = 512
TB = 256
NSLOT = 6144
NBLK = 23
NBLK_PAD = 24

NC, NS = 2, 16
NW = NC * NS
TPW = B // NW


def _router_body(x_ref, wg_ref, bg_ref, slot1_ref, slot2_ref, w1_ref, w2_ref,
                 bexp_ref, nb_ref):
    g = jnp.dot(x_ref[...], wg_ref[...], preferred_element_type=jnp.float32)
    g = g + bg_ref[...]
    iota_e = lax.broadcasted_iota(jnp.int32, (B, E), 1)
    m1 = jnp.max(g, axis=1, keepdims=True)
    a1 = jnp.min(jnp.where(g == m1, iota_e, E), axis=1, keepdims=True)
    gm = jnp.where(iota_e == a1, -jnp.inf, g)
    m2 = jnp.max(gm, axis=1, keepdims=True)
    a2 = jnp.min(jnp.where(gm == m2, iota_e, E), axis=1, keepdims=True)
    w1 = 1.0 / (1.0 + jnp.exp(m2 - m1))
    w1_ref[...] = jnp.broadcast_to(w1, (B, E))
    w2_ref[...] = jnp.broadcast_to(1.0 - w1, (B, E))

    oh1 = (iota_e == a1).astype(jnp.float32)
    oh2 = (iota_e == a2).astype(jnp.float32)
    ib_r = lax.broadcasted_iota(jnp.int32, (B, B), 0)
    ib_c = lax.broadcasted_iota(jnp.int32, (B, B), 1)
    ltri = (ib_r > ib_c).astype(jnp.float32)
    ex1 = jnp.dot(ltri, oh1, preferred_element_type=jnp.float32)
    ex2 = jnp.dot(ltri, oh2, preferred_element_type=jnp.float32)
    counts1 = jnp.sum(oh1, axis=0, keepdims=True)
    counts2 = jnp.sum(oh2, axis=0, keepdims=True)
    counts = counts1 + counts2
    rank1 = jnp.sum(ex1 * oh1, axis=1, keepdims=True)
    rank2 = (jnp.sum(ex2 * oh2, axis=1, keepdims=True)
             + jnp.sum(counts1 * oh2, axis=1, keepdims=True))
    padded = jnp.ceil(counts * (1.0 / TB)) * TB
    ie_r = lax.broadcasted_iota(jnp.int32, (E, E), 0)
    ie_c = lax.broadcasted_iota(jnp.int32, (E, E), 1)
    offs = jnp.sum(jnp.where(ie_r < ie_c, jnp.broadcast_to(padded.reshape(E, 1), (E, E)), 0.0),
                   axis=0, keepdims=True)
    ends = offs + padded
    slot1_ref[...] = (jnp.sum(offs * oh1, axis=1, keepdims=True)
                      + rank1).astype(jnp.int32).reshape(B)
    slot2_ref[...] = (jnp.sum(offs * oh2, axis=1, keepdims=True)
                      + rank2).astype(jnp.int32).reshape(B)
    jblk = (lax.broadcasted_iota(jnp.int32, (NBLK_PAD, E), 0) * TB).astype(jnp.float32)
    bexp = jnp.sum((jnp.broadcast_to(ends, (NBLK_PAD, E)) <= jblk).astype(jnp.int32),
                   axis=1, keepdims=True)
    bexp_ref[...] = jnp.minimum(bexp, E - 1).reshape(NBLK_PAD)
    nb_ref[...] = (jnp.sum(padded, axis=1, keepdims=True) * (1.0 / TB)).astype(jnp.int32)


def _router_call(x, Wg, bg2):
    return pl.pallas_call(
        _router_body,
        out_shape=[
            jax.ShapeDtypeStruct((B,), jnp.int32),
            jax.ShapeDtypeStruct((B,), jnp.int32),
            jax.ShapeDtypeStruct((B, E), jnp.float32),
            jax.ShapeDtypeStruct((B, E), jnp.float32),
            jax.ShapeDtypeStruct((NBLK_PAD,), jnp.int32),
            jax.ShapeDtypeStruct((1, 1), jnp.int32),
        ],
    )(x, Wg, bg2)


def _dispatch_body(x_hbm, slot1_hbm, slot2_hbm, xs_hbm, idx1_v, idx2_v,
                   rows_v, sem1, sem2):
    wid = lax.axis_index("s") * NC + lax.axis_index("c")
    base = wid * TPW
    pltpu.sync_copy(slot1_hbm.at[pl.ds(base, TPW)], idx1_v)
    pltpu.sync_copy(slot2_hbm.at[pl.ds(base, TPW)], idx2_v)
    pltpu.sync_copy(x_hbm.at[pl.ds(base, TPW)], rows_v)
    d1 = pltpu.async_copy(rows_v, xs_hbm.at[idx1_v], sem1)
    d2 = pltpu.async_copy(rows_v, xs_hbm.at[idx2_v], sem2)
    d1.wait()
    d2.wait()


@functools.cache
def _dispatch_call():
    mesh = plsc.VectorSubcoreMesh(core_axis_name="c", subcore_axis_name="s",
                                  num_cores=NC, num_subcores=NS)
    return pl.kernel(
        _dispatch_body,
        out_type=jax.ShapeDtypeStruct((NSLOT, S), jnp.float32),
        mesh=mesh,
        scratch_types=[
            pltpu.VMEM((TPW,), jnp.int32),
            pltpu.VMEM((TPW,), jnp.int32),
            pltpu.VMEM((TPW, S), jnp.float32),
            pltpu.SemaphoreType.DMA,
            pltpu.SemaphoreType.DMA,
        ],
    )


def _mm_body(nb_ref, bexp_ref, xs_ref, we_ref, be_ref, y_ref):
    j = pl.program_id(0)

    @pl.when(j < nb_ref[0])
    def _():
        y_ref[...] = (jnp.dot(xs_ref[...], we_ref[0],
                              preferred_element_type=jnp.float32)
                      + be_ref[0])


def _mm_call(nb, bexp, xs, We, be):
    def _j(j, nb_ref):
        return jnp.minimum(j, nb_ref[0] - 1)

    grid_spec = pltpu.PrefetchScalarGridSpec(
        num_scalar_prefetch=2,
        grid=(NBLK,),
        in_specs=[
            pl.BlockSpec((TB, S), lambda j, nb, bexp: (_j(j, nb), 0)),
            pl.BlockSpec((1, S, P), lambda j, nb, bexp: (bexp[_j(j, nb)], 0, 0)),
            pl.BlockSpec((1, 1, P), lambda j, nb, bexp: (bexp[_j(j, nb)], 0, 0)),
        ],
        out_specs=pl.BlockSpec((TB, P), lambda j, nb, bexp: (_j(j, nb), 0)),
    )
    return pl.pallas_call(
        _mm_body,
        grid_spec=grid_spec,
        out_shape=jax.ShapeDtypeStruct((NSLOT, P), jnp.float32),
    )(nb, bexp, xs, We, be.reshape(E, 1, P))


def _combine_body(y_hbm, slot1_hbm, slot2_hbm, w1_hbm, w2_hbm, out_hbm,
                  idx1_v, idx2_v, w1_v, w2_v, r1_v, r2_v, o_v,
                  sem1, sem2, sem3, sem4, sem5, sem6):
    wid = lax.axis_index("s") * NC + lax.axis_index("c")
    base = wid * TPW
    pltpu.sync_copy(slot1_hbm.at[pl.ds(base, TPW)], idx1_v)
    pltpu.sync_copy(slot2_hbm.at[pl.ds(base, TPW)], idx2_v)
    pltpu.sync_copy(w1_hbm.at[pl.ds(base, TPW)], w1_v)
    pltpu.sync_copy(w2_hbm.at[pl.ds(base, TPW)], w2_v)
    half = TPW // 2
    g1a = pltpu.async_copy(y_hbm.at[idx1_v.at[pl.ds(0, half)]],
                           r1_v.at[pl.ds(0, half)], sem1)
    g2a = pltpu.async_copy(y_hbm.at[idx2_v.at[pl.ds(0, half)]],
                           r2_v.at[pl.ds(0, half)], sem2)
    g1b = pltpu.async_copy(y_hbm.at[idx1_v.at[pl.ds(half, half)]],
                           r1_v.at[pl.ds(half, half)], sem3)
    g2b = pltpu.async_copy(y_hbm.at[idx2_v.at[pl.ds(half, half)]],
                           r2_v.at[pl.ds(half, half)], sem4)

    def body(i, carry):
        w1b = w1_v[i, :]
        w2b = w2_v[i, :]
        for c in range(P // 16):
            sl = pl.ds(c * 16, 16)
            o_v[i, sl] = w1b * r1_v[i, sl] + w2b * r2_v[i, sl]
        return carry

    g1a.wait()
    g2a.wait()
    lax.fori_loop(0, half, body, 0)
    sa = pltpu.async_copy(o_v.at[pl.ds(0, half)],
                          out_hbm.at[pl.ds(base, half)], sem5)
    g1b.wait()
    g2b.wait()
    lax.fori_loop(half, TPW, body, 0)
    sb = pltpu.async_copy(o_v.at[pl.ds(half, half)],
                          out_hbm.at[pl.ds(base + half, half)], sem6)
    sa.wait()
    sb.wait()


@functools.cache
def _combine_call():
    mesh = plsc.VectorSubcoreMesh(core_axis_name="c", subcore_axis_name="s",
                                  num_cores=NC, num_subcores=NS)
    return pl.kernel(
        _combine_body,
        out_type=jax.ShapeDtypeStruct((B, P), jnp.float32),
        mesh=mesh,
        scratch_types=[
            pltpu.VMEM((TPW,), jnp.int32),
            pltpu.VMEM((TPW,), jnp.int32),
            pltpu.VMEM((TPW, E), jnp.float32),
            pltpu.VMEM((TPW, E), jnp.float32),
            pltpu.VMEM((TPW, P), jnp.float32),
            pltpu.VMEM((TPW, P), jnp.float32),
            pltpu.VMEM((TPW, P), jnp.float32),
            pltpu.SemaphoreType.DMA,
            pltpu.SemaphoreType.DMA,
            pltpu.SemaphoreType.DMA,
            pltpu.SemaphoreType.DMA,
            pltpu.SemaphoreType.DMA,
            pltpu.SemaphoreType.DMA,
        ],
    )


def kernel(x, Wg, bg, We, be):
    slot1, slot2, w1c, w2c, bexp, nbc = _router_call(x, Wg, bg.reshape(1, E))
    nb = nbc.reshape(1)
    xs = _dispatch_call()(x, slot1, slot2)
    ys = _mm_call(nb, bexp, xs, We, be)
    return _combine_call()(ys, slot1, slot2, w1c, w2c)
